# Initial kernel scaffold; baseline (speedup 1.0000x reference)
import jax
import jax.numpy as jnp
from jax.experimental import pallas as pl


def kernel(feat, pos_emb):
    raise NotImplementedError("write your pallas kernel here")


# speedup vs baseline: 1.7266x; 1.7266x over previous
import jax
import jax.numpy as jnp
from jax.experimental import pallas as pl

_BS = 512


def _add_body(feat_ref, pos_ref, out_ref):
    out_ref[...] = feat_ref[...] + pos_ref[...][None, :, :]


def kernel(feat, pos_emb):
    B, S, D = feat.shape
    pe = pos_emb[:S]
    grid = (S // _BS,)
    return pl.pallas_call(
        _add_body,
        grid=grid,
        in_specs=[
            pl.BlockSpec((B, _BS, D), lambda i: (0, i, 0)),
            pl.BlockSpec((_BS, D), lambda i: (i, 0)),
        ],
        out_specs=pl.BlockSpec((B, _BS, D), lambda i: (0, i, 0)),
        out_shape=jax.ShapeDtypeStruct((B, S, D), feat.dtype),
    )(feat, pe)
